# Initial kernel scaffold; baseline (speedup 1.0000x reference)
import jax
import jax.numpy as jnp
from jax.experimental import pallas as pl


def kernel(x, edge_index, W1, b1, W2, b2, Wl, bl):
    raise NotImplementedError("write your pallas kernel here")


# speedup vs baseline: 6.7275x; 6.7275x over previous
import functools

import jax
import jax.numpy as jnp
from jax import lax
from jax.experimental import pallas as pl
from jax.experimental.pallas import tpu as pltpu
from jax.experimental.pallas import tpu_sc as plsc

N = 10000
E = 160000
CHW = 128
NS = 16
CHUNK = 128
E_PAD = 163840
IDX_ROWS = E_PAD // CHUNK
AGG_ROWS_PER_TILE = IDX_ROWS // NS
DEG_ROWS_PER_TILE = IDX_ROWS // (2 * NS)
ACC_ROWS = 10240
TRASH = 10100
BLK = 1000


def _memset_rows(ref, n_rows, n_cols, value):
    vec = jnp.full((16,), value, jnp.float32)

    def body(i, _):
        for j in range(n_cols // 16):
            ref[i, pl.ds(j * 16, 16)] = vec
        return 0

    lax.fori_loop(0, n_rows, body, 0)


def _deg_body(dst2d, out, idx_v, ones_v, acc, sem):
    c = lax.axis_index("c")
    t = lax.axis_index("s")
    wid = c * NS + t

    zrows = ACC_ROWS // NS
    _memset_rows(ones_v, CHUNK, CHW, 0.0)
    for j in range(zrows // CHUNK):
        pltpu.sync_copy(ones_v, acc.at[pl.ds(t * zrows + j * CHUNK, CHUNK)])
    _memset_rows(ones_v, CHUNK, CHW, 1.0)
    plsc.subcore_barrier()

    pltpu.sync_copy(dst2d.at[pl.ds(wid * DEG_ROWS_PER_TILE, DEG_ROWS_PER_TILE)],
                    idx_v)

    def body(i, _):
        pltpu.sync_copy(ones_v, acc.at[idx_v.at[i]], add=True)
        return 0

    lax.fori_loop(0, DEG_ROWS_PER_TILE, body, 0)
    plsc.subcore_barrier()

    pltpu.sync_copy(acc.at[pl.ds(t * zrows, zrows)],
                    out.at[c, pl.ds(t * zrows, zrows)])


@functools.partial(jax.jit)
def _deg_call(dst2d):
    mesh = plsc.VectorSubcoreMesh(core_axis_name="c", subcore_axis_name="s", num_cores=2, num_subcores=NS)
    f = pl.kernel(
        _deg_body,
        out_type=jax.ShapeDtypeStruct((2, ACC_ROWS, CHW), jnp.float32),
        mesh=mesh,
        scratch_types=[
            pltpu.VMEM((DEG_ROWS_PER_TILE, CHUNK), jnp.int32),
            pltpu.VMEM((CHUNK, CHW), jnp.float32),
            pltpu.VMEM_SHARED((ACC_ROWS, CHW), jnp.float32),
            pltpu.SemaphoreType.DMA,
        ],
    )
    return f(dst2d)


def _agg_body(s0, s1, src2d, dst2d, out0, out1,
              idxs_v, idxd_v, rows_v, acc, sem):
    c = lax.axis_index("c")
    t = lax.axis_index("s")

    _memset_rows(rows_v, CHUNK, CHW, 0.0)
    zrows = ACC_ROWS // NS
    for j in range(zrows // CHUNK):
        pltpu.sync_copy(rows_v, acc.at[pl.ds(t * zrows + j * CHUNK, CHUNK)])

    pltpu.sync_copy(src2d.at[pl.ds(t * AGG_ROWS_PER_TILE, AGG_ROWS_PER_TILE)],
                    idxs_v)
    pltpu.sync_copy(dst2d.at[pl.ds(t * AGG_ROWS_PER_TILE, AGG_ROWS_PER_TILE)],
                    idxd_v)
    plsc.subcore_barrier()

    def run(s_hbm, out_hbm):
        def body(i, _):
            pltpu.async_copy(s_hbm.at[idxs_v.at[i]], rows_v, sem).wait()
            pltpu.sync_copy(rows_v, acc.at[idxd_v.at[i]], add=True)
            return 0

        lax.fori_loop(0, AGG_ROWS_PER_TILE, body, 0)
        plsc.subcore_barrier()
        orows = ACC_ROWS // NS
        pltpu.sync_copy(acc.at[pl.ds(t * orows, orows)],
                        out_hbm.at[pl.ds(t * orows, orows)])

    @pl.when(c == 0)
    def _():
        run(s0, out0)

    @pl.when(c == 1)
    def _():
        run(s1, out1)


@functools.partial(jax.jit)
def _agg_call(s0, s1, src2d, dst2d):
    mesh = plsc.VectorSubcoreMesh(core_axis_name="c", subcore_axis_name="s", num_cores=2, num_subcores=NS)
    f = pl.kernel(
        _agg_body,
        out_type=[jax.ShapeDtypeStruct((ACC_ROWS, CHW), jnp.float32),
                  jax.ShapeDtypeStruct((ACC_ROWS, CHW), jnp.float32)],
        mesh=mesh,
        scratch_types=[
            pltpu.VMEM((AGG_ROWS_PER_TILE, CHUNK), jnp.int32),
            pltpu.VMEM((AGG_ROWS_PER_TILE, CHUNK), jnp.int32),
            pltpu.VMEM((CHUNK, CHW), jnp.float32),
            pltpu.VMEM_SHARED((ACC_ROWS, CHW), jnp.float32),
            pltpu.SemaphoreType.DMA,
        ],
    )
    return f(s0, s1, src2d, dst2d)


def _dis(deg0_ref, deg1_ref):
    deg = deg0_ref[:, 0] + deg1_ref[:, 0] + 1.0
    return lax.rsqrt(deg)


def _t1_body(deg0, deg1, x_ref, w_ref, s0_ref, s1_ref):
    dis = _dis(deg0, deg1)
    h = jnp.dot(x_ref[...], w_ref[...], preferred_element_type=jnp.float32)
    s = h * dis[:, None]
    s0_ref[...] = s[:, :CHW]
    s1_ref[...] = s[:, CHW:]


def _t2_body(deg0, deg1, a0, a1, s0, s1, b_ref, w_ref, o0_ref, o1_ref):
    dis = _dis(deg0, deg1)
    h0 = jnp.maximum((a0[...] + s0[...]) * dis[:, None] + b_ref[0, :CHW], 0.0)
    h1 = jnp.maximum((a1[...] + s1[...]) * dis[:, None] + b_ref[0, CHW:], 0.0)
    h = jnp.concatenate([h0, h1], axis=1)
    s2 = jnp.dot(h, w_ref[...], preferred_element_type=jnp.float32)
    s2 = s2 * dis[:, None]
    o0_ref[...] = s2[:, :CHW]
    o1_ref[...] = s2[:, CHW:]


def _t3_body(deg0, deg1, a0, a1, s0, s1, b_ref, w_ref, bl_ref, out_ref):
    dis = _dis(deg0, deg1)
    h0 = jnp.maximum((a0[...] + s0[...]) * dis[:, None] + b_ref[0, :CHW], 0.0)
    h1 = jnp.maximum((a1[...] + s1[...]) * dis[:, None] + b_ref[0, CHW:], 0.0)
    h = jnp.concatenate([h0, h1], axis=1)
    out_ref[...] = (jnp.dot(h, w_ref[...], preferred_element_type=jnp.float32)
                    + bl_ref[0, :])


def _row_spec(cols):
    return pl.BlockSpec((BLK, cols), lambda i: (i, 0))


def _full_spec(shape):
    return pl.BlockSpec(shape, lambda i: tuple(0 for _ in shape))


def _t1_call(deg0, deg1, x, w1):
    return pl.pallas_call(
        _t1_body,
        grid=(N // BLK,),
        in_specs=[_row_spec(CHW), _row_spec(CHW), _row_spec(256),
                  _full_spec((256, 256))],
        out_specs=[_row_spec(CHW), _row_spec(CHW)],
        out_shape=[jax.ShapeDtypeStruct((N, CHW), jnp.float32),
                   jax.ShapeDtypeStruct((N, CHW), jnp.float32)],
    )(deg0, deg1, x, w1)


def _t2_call(deg0, deg1, a0, a1, s0, s1, b, w):
    return pl.pallas_call(
        _t2_body,
        grid=(N // BLK,),
        in_specs=[_row_spec(CHW), _row_spec(CHW),
                  _row_spec(CHW), _row_spec(CHW),
                  _row_spec(CHW), _row_spec(CHW),
                  _full_spec((1, 256)), _full_spec((256, 256))],
        out_specs=[_row_spec(CHW), _row_spec(CHW)],
        out_shape=[jax.ShapeDtypeStruct((N, CHW), jnp.float32),
                   jax.ShapeDtypeStruct((N, CHW), jnp.float32)],
    )(deg0, deg1, a0, a1, s0, s1, b, w)


def _t3_call(deg0, deg1, a0, a1, s0, s1, b, w, bl):
    return pl.pallas_call(
        _t3_body,
        grid=(N // BLK,),
        in_specs=[_row_spec(CHW), _row_spec(CHW),
                  _row_spec(CHW), _row_spec(CHW),
                  _row_spec(CHW), _row_spec(CHW),
                  _full_spec((1, 256)), _full_spec((256, CHW)),
                  _full_spec((1, CHW))],
        out_specs=_row_spec(CHW),
        out_shape=jax.ShapeDtypeStruct((N, CHW), jnp.float32),
    )(deg0, deg1, a0, a1, s0, s1, b, w, bl)


def kernel(x, edge_index, W1, b1, W2, b2, Wl, bl):
    src = edge_index[0].astype(jnp.int32)
    dst = edge_index[1].astype(jnp.int32)
    pad = E_PAD - E
    src2d = jnp.concatenate(
        [src, jnp.zeros((pad,), jnp.int32)]).reshape(IDX_ROWS, CHUNK)
    dst2d = jnp.concatenate(
        [dst, jnp.full((pad,), TRASH, jnp.int32)]).reshape(IDX_ROWS, CHUNK)

    degp = _deg_call(dst2d)
    deg0 = degp[0, :N]
    deg1 = degp[1, :N]

    s0, s1 = _t1_call(deg0, deg1, x, W1)
    a0, a1 = _agg_call(s0, s1, src2d, dst2d)
    s2_0, s2_1 = _t2_call(deg0, deg1, a0, a1, s0, s1,
                          b1.reshape(1, -1), W2)
    a2_0, a2_1 = _agg_call(s2_0, s2_1, src2d, dst2d)
    out = _t3_call(deg0, deg1, a2_0, a2_1, s2_0, s2_1,
                   b2.reshape(1, -1), Wl, bl.reshape(1, -1))
    return out


# speedup vs baseline: 7.4909x; 1.1135x over previous
import functools

import jax
import jax.numpy as jnp
from jax import lax
from jax.experimental import pallas as pl
from jax.experimental.pallas import tpu as pltpu
from jax.experimental.pallas import tpu_sc as plsc

N = 10000
E = 160000
CHW = 128
NS = 16
CHUNK = 128
E_PAD = 163840
IDX_ROWS = E_PAD // CHUNK
AGG_ROWS_PER_TILE = IDX_ROWS // NS
DEG_ROWS_PER_TILE = IDX_ROWS // (2 * NS)
ACC_ROWS = 10240
TRASH = 10100
BLK = 1000


def _memset_rows(ref, n_rows, n_cols, value):
    vec = jnp.full((16,), value, jnp.float32)

    def body(i, _):
        for j in range(n_cols // 16):
            ref[i, pl.ds(j * 16, 16)] = vec
        return 0

    lax.fori_loop(0, n_rows, body, 0)


def _deg_body(dst2d, out, idx_v, ones_v, acc, sem):
    c = lax.axis_index("c")
    t = lax.axis_index("s")
    wid = c * NS + t

    zrows = ACC_ROWS // NS
    _memset_rows(ones_v, CHUNK, CHW, 0.0)
    for j in range(zrows // CHUNK):
        pltpu.sync_copy(ones_v, acc.at[pl.ds(t * zrows + j * CHUNK, CHUNK)])
    _memset_rows(ones_v, CHUNK, CHW, 1.0)
    plsc.subcore_barrier()

    pltpu.sync_copy(dst2d.at[pl.ds(wid * DEG_ROWS_PER_TILE, DEG_ROWS_PER_TILE)],
                    idx_v)

    def body(i, _):
        pltpu.sync_copy(ones_v, acc.at[idx_v.at[i]], add=True)
        return 0

    lax.fori_loop(0, DEG_ROWS_PER_TILE, body, 0)
    plsc.subcore_barrier()

    pltpu.sync_copy(acc.at[pl.ds(t * zrows, zrows)],
                    out.at[c, pl.ds(t * zrows, zrows)])


@functools.partial(jax.jit)
def _deg_call(dst2d):
    mesh = plsc.VectorSubcoreMesh(core_axis_name="c", subcore_axis_name="s", num_cores=2, num_subcores=NS)
    f = pl.kernel(
        _deg_body,
        out_type=jax.ShapeDtypeStruct((2, ACC_ROWS, CHW), jnp.float32),
        mesh=mesh,
        scratch_types=[
            pltpu.VMEM((DEG_ROWS_PER_TILE, CHUNK), jnp.int32),
            pltpu.VMEM((CHUNK, CHW), jnp.float32),
            pltpu.VMEM_SHARED((ACC_ROWS, CHW), jnp.float32),
            pltpu.SemaphoreType.DMA,
        ],
    )
    return f(dst2d)


IDX_PHASES = 2
PHASE_ROWS = AGG_ROWS_PER_TILE // IDX_PHASES


def _agg_body(s0, s1, src2d, dst2d, out0, out1,
              idxs_v, idxd_v, rows_v, acc, gsem, ssem):
    c = lax.axis_index("c")
    t = lax.axis_index("s")

    _memset_rows(rows_v.at[0], CHUNK, CHW, 0.0)
    zrows = ACC_ROWS // NS
    for j in range(zrows // CHUNK):
        pltpu.sync_copy(rows_v.at[0], acc.at[pl.ds(t * zrows + j * CHUNK, CHUNK)])
    plsc.subcore_barrier()

    def run(s_hbm, out_hbm):
        def gstart(k, b):
            pltpu.async_copy(s_hbm.at[idxs_v.at[k]], rows_v.at[b], gsem)

        def gwait(b):
            pltpu.make_async_copy(s_hbm.at[idxs_v.at[0]], rows_v.at[b],
                                  gsem).wait()

        def sstart(k, b):
            pltpu.async_copy(rows_v.at[b], acc.at[idxd_v.at[k]], ssem,
                             add=True)

        def swait(b):
            pltpu.make_async_copy(rows_v.at[b], acc.at[idxd_v.at[0]],
                                  ssem).wait()

        for p in range(IDX_PHASES):
            base = t * AGG_ROWS_PER_TILE + p * PHASE_ROWS
            pltpu.sync_copy(src2d.at[pl.ds(base, PHASE_ROWS)], idxs_v)
            pltpu.sync_copy(dst2d.at[pl.ds(base, PHASE_ROWS)], idxd_v)
            gstart(0, 0)

            def lbody(j, _):
                for b in range(2):
                    k = j * 2 + b
                    gwait(b)
                    sstart(k, b)

                    @pl.when(k > 0)
                    def _():
                        swait(1 - b)

                    @pl.when(k < PHASE_ROWS - 1)
                    def _():
                        gstart(k + 1, 1 - b)
                return 0

            lax.fori_loop(0, PHASE_ROWS // 2, lbody, 0)
            swait(1)
        plsc.subcore_barrier()
        orows = ACC_ROWS // NS
        pltpu.sync_copy(acc.at[pl.ds(t * orows, orows)],
                        out_hbm.at[pl.ds(t * orows, orows)])

    @pl.when(c == 0)
    def _():
        run(s0, out0)

    @pl.when(c == 1)
    def _():
        run(s1, out1)


@functools.partial(jax.jit)
def _agg_call(s0, s1, src2d, dst2d):
    mesh = plsc.VectorSubcoreMesh(core_axis_name="c", subcore_axis_name="s", num_cores=2, num_subcores=NS)
    f = pl.kernel(
        _agg_body,
        out_type=[jax.ShapeDtypeStruct((ACC_ROWS, CHW), jnp.float32),
                  jax.ShapeDtypeStruct((ACC_ROWS, CHW), jnp.float32)],
        mesh=mesh,
        scratch_types=[
            pltpu.VMEM((PHASE_ROWS, CHUNK), jnp.int32),
            pltpu.VMEM((PHASE_ROWS, CHUNK), jnp.int32),
            pltpu.VMEM((2, CHUNK, CHW), jnp.float32),
            pltpu.VMEM_SHARED((ACC_ROWS, CHW), jnp.float32),
            pltpu.SemaphoreType.DMA,
            pltpu.SemaphoreType.DMA,
        ],
    )
    return f(s0, s1, src2d, dst2d)


def _dis(deg0_ref, deg1_ref):
    deg = deg0_ref[:, 0] + deg1_ref[:, 0] + 1.0
    return lax.rsqrt(deg)


def _t1_body(deg0, deg1, x_ref, w_ref, s0_ref, s1_ref):
    dis = _dis(deg0, deg1)
    h = jnp.dot(x_ref[...], w_ref[...], preferred_element_type=jnp.float32)
    s = h * dis[:, None]
    s0_ref[...] = s[:, :CHW]
    s1_ref[...] = s[:, CHW:]


def _t2_body(deg0, deg1, a0, a1, s0, s1, b_ref, w_ref, o0_ref, o1_ref):
    dis = _dis(deg0, deg1)
    h0 = jnp.maximum((a0[...] + s0[...]) * dis[:, None] + b_ref[0, :CHW], 0.0)
    h1 = jnp.maximum((a1[...] + s1[...]) * dis[:, None] + b_ref[0, CHW:], 0.0)
    h = jnp.concatenate([h0, h1], axis=1)
    s2 = jnp.dot(h, w_ref[...], preferred_element_type=jnp.float32)
    s2 = s2 * dis[:, None]
    o0_ref[...] = s2[:, :CHW]
    o1_ref[...] = s2[:, CHW:]


def _t3_body(deg0, deg1, a0, a1, s0, s1, b_ref, w_ref, bl_ref, out_ref):
    dis = _dis(deg0, deg1)
    h0 = jnp.maximum((a0[...] + s0[...]) * dis[:, None] + b_ref[0, :CHW], 0.0)
    h1 = jnp.maximum((a1[...] + s1[...]) * dis[:, None] + b_ref[0, CHW:], 0.0)
    h = jnp.concatenate([h0, h1], axis=1)
    out_ref[...] = (jnp.dot(h, w_ref[...], preferred_element_type=jnp.float32)
                    + bl_ref[0, :])


def _row_spec(cols):
    return pl.BlockSpec((BLK, cols), lambda i: (i, 0))


def _full_spec(shape):
    return pl.BlockSpec(shape, lambda i: tuple(0 for _ in shape))


def _t1_call(deg0, deg1, x, w1):
    return pl.pallas_call(
        _t1_body,
        grid=(N // BLK,),
        in_specs=[_row_spec(CHW), _row_spec(CHW), _row_spec(256),
                  _full_spec((256, 256))],
        out_specs=[_row_spec(CHW), _row_spec(CHW)],
        out_shape=[jax.ShapeDtypeStruct((N, CHW), jnp.float32),
                   jax.ShapeDtypeStruct((N, CHW), jnp.float32)],
    )(deg0, deg1, x, w1)


def _t2_call(deg0, deg1, a0, a1, s0, s1, b, w):
    return pl.pallas_call(
        _t2_body,
        grid=(N // BLK,),
        in_specs=[_row_spec(CHW), _row_spec(CHW),
                  _row_spec(CHW), _row_spec(CHW),
                  _row_spec(CHW), _row_spec(CHW),
                  _full_spec((1, 256)), _full_spec((256, 256))],
        out_specs=[_row_spec(CHW), _row_spec(CHW)],
        out_shape=[jax.ShapeDtypeStruct((N, CHW), jnp.float32),
                   jax.ShapeDtypeStruct((N, CHW), jnp.float32)],
    )(deg0, deg1, a0, a1, s0, s1, b, w)


def _t3_call(deg0, deg1, a0, a1, s0, s1, b, w, bl):
    return pl.pallas_call(
        _t3_body,
        grid=(N // BLK,),
        in_specs=[_row_spec(CHW), _row_spec(CHW),
                  _row_spec(CHW), _row_spec(CHW),
                  _row_spec(CHW), _row_spec(CHW),
                  _full_spec((1, 256)), _full_spec((256, CHW)),
                  _full_spec((1, CHW))],
        out_specs=_row_spec(CHW),
        out_shape=jax.ShapeDtypeStruct((N, CHW), jnp.float32),
    )(deg0, deg1, a0, a1, s0, s1, b, w, bl)


def kernel(x, edge_index, W1, b1, W2, b2, Wl, bl):
    src = edge_index[0].astype(jnp.int32)
    dst = edge_index[1].astype(jnp.int32)
    pad = E_PAD - E
    src2d = jnp.concatenate(
        [src, jnp.zeros((pad,), jnp.int32)]).reshape(IDX_ROWS, CHUNK)
    dst2d = jnp.concatenate(
        [dst, jnp.full((pad,), TRASH, jnp.int32)]).reshape(IDX_ROWS, CHUNK)

    degp = _deg_call(dst2d)
    deg0 = degp[0, :N]
    deg1 = degp[1, :N]

    s0, s1 = _t1_call(deg0, deg1, x, W1)
    a0, a1 = _agg_call(s0, s1, src2d, dst2d)
    s2_0, s2_1 = _t2_call(deg0, deg1, a0, a1, s0, s1,
                          b1.reshape(1, -1), W2)
    a2_0, a2_1 = _agg_call(s2_0, s2_1, src2d, dst2d)
    out = _t3_call(deg0, deg1, a2_0, a2_1, s2_0, s2_1,
                   b2.reshape(1, -1), Wl, bl.reshape(1, -1))
    return out


# speedup vs baseline: 8.2245x; 1.0979x over previous
import functools

import jax
import jax.numpy as jnp
from jax import lax
from jax.experimental import pallas as pl
from jax.experimental.pallas import tpu as pltpu
from jax.experimental.pallas import tpu_sc as plsc

N = 10000
E = 160000
CHW = 128
NS = 16
CHUNK = 64
E_PAD = 163840
IDX_ROWS = E_PAD // CHUNK
AGG_ROWS_PER_TILE = IDX_ROWS // NS
DEG_ROWS_PER_TILE = IDX_ROWS // (2 * NS)
ACC_ROWS = 10240
TRASH = 10100
BLK = 1000


def _memset_rows(ref, n_rows, n_cols, value):
    vec = jnp.full((16,), value, jnp.float32)

    def body(i, _):
        for j in range(n_cols // 16):
            ref[i, pl.ds(j * 16, 16)] = vec
        return 0

    lax.fori_loop(0, n_rows, body, 0)


def _deg_body(dst2d, out, idx_v, ones_v, acc, sem):
    c = lax.axis_index("c")
    t = lax.axis_index("s")
    wid = c * NS + t

    zrows = ACC_ROWS // NS
    _memset_rows(ones_v, CHUNK, CHW, 0.0)
    for j in range(zrows // CHUNK):
        pltpu.sync_copy(ones_v, acc.at[pl.ds(t * zrows + j * CHUNK, CHUNK)])
    _memset_rows(ones_v, CHUNK, CHW, 1.0)
    plsc.subcore_barrier()

    pltpu.sync_copy(dst2d.at[pl.ds(wid * DEG_ROWS_PER_TILE, DEG_ROWS_PER_TILE)],
                    idx_v)

    def body(i, _):
        pltpu.async_copy(ones_v, acc.at[idx_v.at[i]], sem, add=True)
        return 0

    lax.fori_loop(0, DEG_ROWS_PER_TILE, body, 0)

    def drain(i, _):
        pltpu.make_async_copy(ones_v, acc.at[idx_v.at[0]], sem).wait()
        return 0

    lax.fori_loop(0, DEG_ROWS_PER_TILE, drain, 0)
    plsc.subcore_barrier()

    pltpu.sync_copy(acc.at[pl.ds(t * zrows, zrows)],
                    out.at[c, pl.ds(t * zrows, zrows)])


@functools.partial(jax.jit)
def _deg_call(dst2d):
    mesh = plsc.VectorSubcoreMesh(core_axis_name="c", subcore_axis_name="s", num_cores=2, num_subcores=NS)
    f = pl.kernel(
        _deg_body,
        out_type=jax.ShapeDtypeStruct((2, ACC_ROWS, CHW), jnp.float32),
        mesh=mesh,
        scratch_types=[
            pltpu.VMEM((DEG_ROWS_PER_TILE, CHUNK), jnp.int32),
            pltpu.VMEM((CHUNK, CHW), jnp.float32),
            pltpu.VMEM_SHARED((ACC_ROWS, CHW), jnp.float32),
            pltpu.SemaphoreType.DMA,
        ],
    )
    return f(dst2d)


IDX_PHASES = 4
PHASE_ROWS = AGG_ROWS_PER_TILE // IDX_PHASES
NBUF = 4


def _agg_body(s0, s1, src2d, dst2d, out0, out1,
              idxs_v, idxd_v, rows_v, acc, gsem, ssem):
    c = lax.axis_index("c")
    t = lax.axis_index("s")

    _memset_rows(rows_v.at[0], CHUNK, CHW, 0.0)
    zrows = ACC_ROWS // NS
    for j in range(zrows // CHUNK):
        pltpu.sync_copy(rows_v.at[0], acc.at[pl.ds(t * zrows + j * CHUNK, CHUNK)])
    plsc.subcore_barrier()

    def run(s_hbm, out_hbm):
        def gstart(k, b):
            pltpu.async_copy(s_hbm.at[idxs_v.at[k]], rows_v.at[b], gsem)

        def gwait(b):
            pltpu.make_async_copy(s_hbm.at[idxs_v.at[0]], rows_v.at[b],
                                  gsem).wait()

        def sstart(k, b):
            pltpu.async_copy(rows_v.at[b], acc.at[idxd_v.at[k]], ssem,
                             add=True)

        def swait(b):
            pltpu.make_async_copy(rows_v.at[b], acc.at[idxd_v.at[0]],
                                  ssem).wait()

        for p in range(IDX_PHASES):
            base = t * AGG_ROWS_PER_TILE + p * PHASE_ROWS
            pltpu.sync_copy(src2d.at[pl.ds(base, PHASE_ROWS)], idxs_v)
            pltpu.sync_copy(dst2d.at[pl.ds(base, PHASE_ROWS)], idxd_v)
            gstart(0, 0)
            gstart(1, 1)

            def lbody(j, _):
                for q in range(NBUF):
                    k = j * NBUF + q
                    gwait(q)
                    sstart(k, q)

                    @pl.when(k >= 2)
                    def _():
                        swait((q + 2) % NBUF)

                    @pl.when(k + 2 < PHASE_ROWS)
                    def _():
                        gstart(k + 2, (q + 2) % NBUF)
                return 0

            lax.fori_loop(0, PHASE_ROWS // NBUF, lbody, 0)
            swait((PHASE_ROWS - 2) % NBUF)
            swait((PHASE_ROWS - 1) % NBUF)
        plsc.subcore_barrier()
        orows = ACC_ROWS // NS
        pltpu.sync_copy(acc.at[pl.ds(t * orows, orows)],
                        out_hbm.at[pl.ds(t * orows, orows)])

    @pl.when(c == 0)
    def _():
        run(s0, out0)

    @pl.when(c == 1)
    def _():
        run(s1, out1)


@functools.partial(jax.jit)
def _agg_call(s0, s1, src2d, dst2d):
    mesh = plsc.VectorSubcoreMesh(core_axis_name="c", subcore_axis_name="s", num_cores=2, num_subcores=NS)
    f = pl.kernel(
        _agg_body,
        out_type=[jax.ShapeDtypeStruct((ACC_ROWS, CHW), jnp.float32),
                  jax.ShapeDtypeStruct((ACC_ROWS, CHW), jnp.float32)],
        mesh=mesh,
        scratch_types=[
            pltpu.VMEM((PHASE_ROWS, CHUNK), jnp.int32),
            pltpu.VMEM((PHASE_ROWS, CHUNK), jnp.int32),
            pltpu.VMEM((NBUF, CHUNK, CHW), jnp.float32),
            pltpu.VMEM_SHARED((ACC_ROWS, CHW), jnp.float32),
            pltpu.SemaphoreType.DMA,
            pltpu.SemaphoreType.DMA,
        ],
    )
    return f(s0, s1, src2d, dst2d)


def _dis(deg0_ref, deg1_ref):
    deg = deg0_ref[:, 0] + deg1_ref[:, 0] + 1.0
    return lax.rsqrt(deg)


def _t1_body(deg0, deg1, x_ref, w_ref, s0_ref, s1_ref):
    dis = _dis(deg0, deg1)
    h = jnp.dot(x_ref[...], w_ref[...], preferred_element_type=jnp.float32)
    s = h * dis[:, None]
    s0_ref[...] = s[:, :CHW]
    s1_ref[...] = s[:, CHW:]


def _t2_body(deg0, deg1, a0, a1, s0, s1, b_ref, w_ref, o0_ref, o1_ref):
    dis = _dis(deg0, deg1)
    h0 = jnp.maximum((a0[...] + s0[...]) * dis[:, None] + b_ref[0, :CHW], 0.0)
    h1 = jnp.maximum((a1[...] + s1[...]) * dis[:, None] + b_ref[0, CHW:], 0.0)
    h = jnp.concatenate([h0, h1], axis=1)
    s2 = jnp.dot(h, w_ref[...], preferred_element_type=jnp.float32)
    s2 = s2 * dis[:, None]
    o0_ref[...] = s2[:, :CHW]
    o1_ref[...] = s2[:, CHW:]


def _t3_body(deg0, deg1, a0, a1, s0, s1, b_ref, w_ref, bl_ref, out_ref):
    dis = _dis(deg0, deg1)
    h0 = jnp.maximum((a0[...] + s0[...]) * dis[:, None] + b_ref[0, :CHW], 0.0)
    h1 = jnp.maximum((a1[...] + s1[...]) * dis[:, None] + b_ref[0, CHW:], 0.0)
    h = jnp.concatenate([h0, h1], axis=1)
    out_ref[...] = (jnp.dot(h, w_ref[...], preferred_element_type=jnp.float32)
                    + bl_ref[0, :])


def _row_spec(cols):
    return pl.BlockSpec((BLK, cols), lambda i: (i, 0))


def _full_spec(shape):
    return pl.BlockSpec(shape, lambda i: tuple(0 for _ in shape))


def _t1_call(deg0, deg1, x, w1):
    return pl.pallas_call(
        _t1_body,
        grid=(N // BLK,),
        in_specs=[_row_spec(CHW), _row_spec(CHW), _row_spec(256),
                  _full_spec((256, 256))],
        out_specs=[_row_spec(CHW), _row_spec(CHW)],
        out_shape=[jax.ShapeDtypeStruct((N, CHW), jnp.float32),
                   jax.ShapeDtypeStruct((N, CHW), jnp.float32)],
    )(deg0, deg1, x, w1)


def _t2_call(deg0, deg1, a0, a1, s0, s1, b, w):
    return pl.pallas_call(
        _t2_body,
        grid=(N // BLK,),
        in_specs=[_row_spec(CHW), _row_spec(CHW),
                  _row_spec(CHW), _row_spec(CHW),
                  _row_spec(CHW), _row_spec(CHW),
                  _full_spec((1, 256)), _full_spec((256, 256))],
        out_specs=[_row_spec(CHW), _row_spec(CHW)],
        out_shape=[jax.ShapeDtypeStruct((N, CHW), jnp.float32),
                   jax.ShapeDtypeStruct((N, CHW), jnp.float32)],
    )(deg0, deg1, a0, a1, s0, s1, b, w)


def _t3_call(deg0, deg1, a0, a1, s0, s1, b, w, bl):
    return pl.pallas_call(
        _t3_body,
        grid=(N // BLK,),
        in_specs=[_row_spec(CHW), _row_spec(CHW),
                  _row_spec(CHW), _row_spec(CHW),
                  _row_spec(CHW), _row_spec(CHW),
                  _full_spec((1, 256)), _full_spec((256, CHW)),
                  _full_spec((1, CHW))],
        out_specs=_row_spec(CHW),
        out_shape=jax.ShapeDtypeStruct((N, CHW), jnp.float32),
    )(deg0, deg1, a0, a1, s0, s1, b, w, bl)


def kernel(x, edge_index, W1, b1, W2, b2, Wl, bl):
    src = edge_index[0].astype(jnp.int32)
    dst = edge_index[1].astype(jnp.int32)
    pad = E_PAD - E
    src2d = jnp.concatenate(
        [src, jnp.zeros((pad,), jnp.int32)]).reshape(IDX_ROWS, CHUNK)
    dst2d = jnp.concatenate(
        [dst, jnp.full((pad,), TRASH, jnp.int32)]).reshape(IDX_ROWS, CHUNK)

    degp = _deg_call(dst2d)
    deg0 = degp[0, :N]
    deg1 = degp[1, :N]

    s0, s1 = _t1_call(deg0, deg1, x, W1)
    a0, a1 = _agg_call(s0, s1, src2d, dst2d)
    s2_0, s2_1 = _t2_call(deg0, deg1, a0, a1, s0, s1,
                          b1.reshape(1, -1), W2)
    a2_0, a2_1 = _agg_call(s2_0, s2_1, src2d, dst2d)
    out = _t3_call(deg0, deg1, a2_0, a2_1, s2_0, s2_1,
                   b2.reshape(1, -1), Wl, bl.reshape(1, -1))
    return out
